# baseline (speedup 1.0000x reference)
import functools

import jax
import jax.numpy as jnp
from jax import lax
from jax.experimental import pallas as pl
from jax.experimental.pallas import tpu as pltpu
from jax.experimental.pallas import tpu_sc as plsc

BATCH, F, S = 2, 768, 8192
E = 64
T = BATCH * S
CAP = T // E
SEQ_BLK = 2048
NBLK = S // SEQ_BLK
NW = 32
RPW = T // NW
CHUNK = 128
NCH = RPW // CHUNK


def _trb(a):
    ident = jnp.eye(a.shape[1], dtype=jnp.float32)
    return lax.dot_general(ident, a, (((1,), (1,)), ((), ())),
                           preferred_element_type=jnp.float32)


def _route_body(gates_ref, idx_ref):
    row_i = lax.broadcasted_iota(jnp.int32, (128, 128), 0)
    col_i = lax.broadcasted_iota(jnp.int32, (128, 128), 1)
    tok = row_i * 128 + col_i
    tok_f = tok.astype(jnp.float32)
    tri_lo = (col_i < row_i).astype(jnp.float32)
    tri_up = (row_i < col_i).astype(jnp.float32)
    r3 = lax.broadcasted_iota(jnp.int32, (CAP, 128, 128), 0).astype(jnp.float32)
    r1 = lax.broadcasted_iota(jnp.int32, (CAP,), 0).astype(jnp.float32)

    def expert_step(e, mask):
        v = gates_ref[e] * mask
        bits = lax.bitcast_convert_type(v, jnp.int32)

        def bs_val(_, c):
            lo, hi = c
            mid = (lo + hi) // 2
            cnt = jnp.sum((bits >= mid).astype(jnp.int32))
            ge = cnt >= CAP
            return jnp.where(ge, mid, lo), jnp.where(ge, hi, mid)

        lo, _ = lax.fori_loop(0, 31, bs_val,
                              (jnp.int32(0), jnp.int32(0x40000000)))
        gt = bits > lo
        tie = bits == lo
        need = CAP - jnp.sum(gt.astype(jnp.int32))

        def bs_tie(_, c):
            l2, h2 = c
            mid = (l2 + h2) // 2
            cnt = jnp.sum((tie & (tok <= mid)).astype(jnp.int32))
            ge = cnt >= need
            return jnp.where(ge, l2, mid), jnp.where(ge, mid, h2)

        _, bnd = lax.fori_loop(0, 14, bs_tie,
                               (jnp.int32(-1), jnp.int32(T - 1)))
        sel = gt | (tie & (tok <= bnd))
        self_f = sel.astype(jnp.float32)
        row_cnt = jnp.sum(self_f, axis=1, keepdims=True)
        row_off = lax.dot_general(tri_lo, row_cnt, (((1,), (0,)), ((), ())),
                                  preferred_element_type=jnp.float32)
        excl = lax.dot_general(self_f, tri_up, (((1,), (0,)), ((), ())),
                               preferred_element_type=jnp.float32)
        pos = row_off + excl
        eq = sel[None, :, :] & (pos[None, :, :] == r3)
        vals_c = jnp.sum(jnp.sum(jnp.where(eq, v[None, :, :], 0.0), axis=2),
                         axis=1)
        ids_c = jnp.sum(jnp.sum(jnp.where(eq, tok_f[None, :, :], 0.0), axis=2),
                        axis=1)
        beats = ((vals_c[None, :] > vals_c[:, None])
                 | ((vals_c[None, :] == vals_c[:, None])
                    & (ids_c[None, :] < ids_c[:, None])))
        rank = jnp.sum(beats.astype(jnp.float32), axis=1)
        ro = rank[None, :] == r1[:, None]
        idx_row = jnp.sum(jnp.where(ro, ids_c[None, :], 0.0), axis=1)
        idx_ref[pl.ds(e, 1), :] = idx_row.astype(jnp.int32)[None, :]
        return mask * (1.0 - self_f)

    lax.fori_loop(0, E, expert_step, jnp.ones((128, 128), jnp.float32))


def _route(gates_t):
    return pl.pallas_call(
        _route_body,
        out_shape=jax.ShapeDtypeStruct((E, CAP), jnp.int32),
    )(gates_t.reshape(E, 128, 128))


def _gather(inp_tm, gates_tm, idx3):
    mesh = plsc.VectorSubcoreMesh(core_axis_name="c", subcore_axis_name="s")

    @functools.partial(
        pl.kernel, mesh=mesh,
        out_type=[jax.ShapeDtypeStruct((T, F), jnp.float32),
                  jax.ShapeDtypeStruct((T, 128), jnp.float32)],
        scratch_types=[pltpu.VMEM((NCH, CHUNK), jnp.int32),
                       pltpu.VMEM((CHUNK, F), jnp.float32),
                       pltpu.VMEM((CHUNK, 128), jnp.float32),
                       pltpu.SemaphoreType.DMA,
                       pltpu.SemaphoreType.DMA],
    )
    def k(inp_hbm, gtm_hbm, idx_hbm, xg_hbm, gg_hbm, idx_v, rows_v, g_v, s1, s2):
        wid = lax.axis_index("s") * 2 + lax.axis_index("c")
        pltpu.sync_copy(idx_hbm.at[wid], idx_v)
        for c in range(NCH):
            base = wid * RPW + c * CHUNK
            a = pltpu.async_copy(inp_hbm.at[idx_v.at[c]], rows_v, s1)
            b = pltpu.async_copy(gtm_hbm.at[idx_v.at[c]], g_v, s2)
            a.wait()
            b.wait()
            pltpu.sync_copy(rows_v, xg_hbm.at[pl.ds(base, CHUNK)])
            pltpu.sync_copy(g_v, gg_hbm.at[pl.ds(base, CHUNK)])

    return k(inp_tm, gates_tm, idx3)


def _expert_body(xg_ref, gg_ref, w_ref, y_ref):
    q = (lax.broadcasted_iota(jnp.int32, (CAP, 128), 0) // 4
         == lax.broadcasted_iota(jnp.int32, (CAP, 128), 1)).astype(jnp.float32)
    scale = jnp.sum(gg_ref[...] * q, axis=1, keepdims=True)
    y_ref[...] = jnp.dot(xg_ref[...] * scale, w_ref[0],
                         preferred_element_type=jnp.float32)


def _expert_mm(xg, gg, expert_w):
    return pl.pallas_call(
        _expert_body,
        grid=(E,),
        in_specs=[
            pl.BlockSpec((CAP, F), lambda e: (e, 0)),
            pl.BlockSpec((CAP, 128), lambda e: (e, 0)),
            pl.BlockSpec((1, F, F), lambda e: (e, 0, 0)),
        ],
        out_specs=pl.BlockSpec((CAP, F), lambda e: (e, 0)),
        out_shape=jax.ShapeDtypeStruct((T, F), jnp.float32),
    )(xg, gg, expert_w)


def _scatter(y, idx3):
    mesh = plsc.VectorSubcoreMesh(core_axis_name="c", subcore_axis_name="s")

    @functools.partial(
        pl.kernel, mesh=mesh,
        out_type=jax.ShapeDtypeStruct((T, F), jnp.float32),
        scratch_types=[pltpu.VMEM((NCH, CHUNK), jnp.int32),
                       pltpu.VMEM((CHUNK, F), jnp.float32),
                       pltpu.SemaphoreType.DMA],
    )
    def k(y_hbm, idx_hbm, out_hbm, idx_v, rows_v, sem):
        wid = lax.axis_index("s") * 2 + lax.axis_index("c")
        pltpu.sync_copy(idx_hbm.at[wid], idx_v)
        for c in range(NCH):
            base = wid * RPW + c * CHUNK
            pltpu.sync_copy(y_hbm.at[pl.ds(base, CHUNK)], rows_v)
            pltpu.async_copy(rows_v, out_hbm.at[idx_v.at[c]], sem).wait()

    return k(y, idx3)


def _untrans_body(y_ref, out_ref):
    out_ref[0] = _trb(y_ref[0])


def _untranspose(y_tm):
    return pl.pallas_call(
        _untrans_body,
        grid=(BATCH, NBLK),
        in_specs=[pl.BlockSpec((1, SEQ_BLK, F), lambda b, j: (b * NBLK + j, 0, 0))],
        out_specs=pl.BlockSpec((1, F, SEQ_BLK), lambda b, j: (b, 0, j)),
        out_shape=jax.ShapeDtypeStruct((BATCH, F, S), jnp.float32),
    )(y_tm.reshape(BATCH * NBLK, SEQ_BLK, F))


def kernel(x, gate, expert_w):
    inp_tm = jnp.transpose(x.astype(jnp.float32), (0, 2, 1)).reshape(T, F)
    logits = inp_tm @ gate.astype(jnp.float32)
    gates = jax.nn.softmax(logits, axis=1)
    gates_pad = jnp.pad(gates, ((0, 0), (0, 128 - E)))
    idx = _route(jnp.transpose(gates))
    idx3 = idx.reshape(NW, NCH, CHUNK)
    xg, gg = _gather(inp_tm, gates_pad, idx3)
    y = _expert_mm(xg, gg, expert_w.astype(jnp.float32))
    y_tm = _scatter(y, idx3)
    return _untranspose(y_tm)


# speedup vs baseline: 1.0503x; 1.0503x over previous
import functools

import jax
import jax.numpy as jnp
from jax import lax
from jax.experimental import pallas as pl
from jax.experimental.pallas import tpu as pltpu
from jax.experimental.pallas import tpu_sc as plsc

BATCH, F, S = 2, 768, 8192
E = 64
T = BATCH * S
CAP = T // E
SEQ_BLK = 2048
NBLK = S // SEQ_BLK
NW = 32
RPW = T // NW
CHUNK = 128
NCH = RPW // CHUNK


def _trb(a):
    ident = jnp.eye(a.shape[1], dtype=jnp.float32)
    return lax.dot_general(ident, a, (((1,), (1,)), ((), ())),
                           preferred_element_type=jnp.float32)


def _phase1_body(gates_ref, owner_ref):
    row_i = lax.broadcasted_iota(jnp.int32, (128, 128), 0)
    col_i = lax.broadcasted_iota(jnp.int32, (128, 128), 1)
    tok = row_i * 128 + col_i

    def expert_step(e, owner):
        v = jnp.where(owner == E, gates_ref[e], 0.0)
        bits = lax.bitcast_convert_type(v, jnp.int32)

        def bs_val(_, c):
            lo, hi = c
            mid = (lo + hi) // 2
            cnt = jnp.sum((bits >= mid).astype(jnp.int32))
            ge = cnt >= CAP
            return jnp.where(ge, mid, lo), jnp.where(ge, hi, mid)

        lo, _ = lax.fori_loop(0, 31, bs_val,
                              (jnp.int32(0), jnp.int32(0x40000000)))
        gt = bits > lo
        tie = bits == lo
        need = CAP - jnp.sum(gt.astype(jnp.int32))
        n_tie = jnp.sum(tie.astype(jnp.int32))

        def bs_tie(_, c):
            l2, h2 = c
            mid = (l2 + h2) // 2
            cnt = jnp.sum((tie & (tok <= mid)).astype(jnp.int32))
            ge = cnt >= need
            return jnp.where(ge, l2, mid), jnp.where(ge, mid, h2)

        bnd = lax.cond(
            n_tie == need, lambda: jnp.int32(T - 1),
            lambda: lax.fori_loop(0, 14, bs_tie,
                                  (jnp.int32(-1), jnp.int32(T - 1)))[1])
        sel = gt | (tie & (tok <= bnd))
        return jnp.where(sel, e, owner)

    owner_ref[...] = lax.fori_loop(
        0, E, expert_step, jnp.full((128, 128), E, jnp.int32))


def _phase2_body(owner_ref, gates_ref, idx_ref):
    e = pl.program_id(0)
    row_i = lax.broadcasted_iota(jnp.int32, (128, 128), 0)
    col_i = lax.broadcasted_iota(jnp.int32, (128, 128), 1)
    tok_f = (row_i * 128 + col_i).astype(jnp.float32)
    tri_lo = (col_i < row_i).astype(jnp.float32)
    tri_up = (row_i < col_i).astype(jnp.float32)
    r3 = lax.broadcasted_iota(jnp.int32, (CAP, 128, 128), 0).astype(jnp.float32)
    r1 = lax.broadcasted_iota(jnp.int32, (CAP,), 0).astype(jnp.float32)

    own = owner_ref[...] == e
    v = jnp.where(own, gates_ref[0], 0.0)
    own_f = own.astype(jnp.float32)
    row_cnt = jnp.sum(own_f, axis=1, keepdims=True)
    row_off = lax.dot_general(tri_lo, row_cnt, (((1,), (0,)), ((), ())),
                              preferred_element_type=jnp.float32)
    excl = lax.dot_general(own_f, tri_up, (((1,), (0,)), ((), ())),
                           preferred_element_type=jnp.float32)
    pos = jnp.where(own, row_off + excl, -1.0)
    eq = pos[None, :, :] == r3
    vals_c = jnp.sum(jnp.sum(jnp.where(eq, v[None, :, :], 0.0), axis=2),
                     axis=1)
    ids_c = jnp.sum(jnp.sum(jnp.where(eq, tok_f[None, :, :], 0.0), axis=2),
                    axis=1)
    beats = ((vals_c[None, :] > vals_c[:, None])
             | ((vals_c[None, :] == vals_c[:, None])
                & (ids_c[None, :] < ids_c[:, None])))
    rank = jnp.sum(beats.astype(jnp.float32), axis=1)
    ro = rank[None, :] == r1[:, None]
    idx_row = jnp.sum(jnp.where(ro, ids_c[None, :], 0.0), axis=1)
    idx_ref[0, 0, :] = idx_row.astype(jnp.int32)


def _route(gates_t):
    gates3 = gates_t.reshape(E, 128, 128)
    owner = pl.pallas_call(
        _phase1_body,
        out_shape=jax.ShapeDtypeStruct((128, 128), jnp.int32),
    )(gates3)
    return pl.pallas_call(
        _phase2_body,
        grid=(E,),
        in_specs=[
            pl.BlockSpec((128, 128), lambda e: (0, 0)),
            pl.BlockSpec((1, 128, 128), lambda e: (e, 0, 0)),
        ],
        out_specs=pl.BlockSpec((1, 1, CAP), lambda e: (e, 0, 0)),
        out_shape=jax.ShapeDtypeStruct((E, 1, CAP), jnp.int32),
    )(owner, gates3).reshape(E, CAP)


def _gather(inp_tm, gates_tm, idx3):
    mesh = plsc.VectorSubcoreMesh(core_axis_name="c", subcore_axis_name="s")

    @functools.partial(
        pl.kernel, mesh=mesh,
        out_type=[jax.ShapeDtypeStruct((T, F), jnp.float32),
                  jax.ShapeDtypeStruct((T, 128), jnp.float32)],
        scratch_types=[pltpu.VMEM((NCH, CHUNK), jnp.int32),
                       pltpu.VMEM((CHUNK, F), jnp.float32),
                       pltpu.VMEM((CHUNK, 128), jnp.float32),
                       pltpu.SemaphoreType.DMA,
                       pltpu.SemaphoreType.DMA],
    )
    def k(inp_hbm, gtm_hbm, idx_hbm, xg_hbm, gg_hbm, idx_v, rows_v, g_v, s1, s2):
        wid = lax.axis_index("s") * 2 + lax.axis_index("c")
        pltpu.sync_copy(idx_hbm.at[wid], idx_v)
        for c in range(NCH):
            base = wid * RPW + c * CHUNK
            a = pltpu.async_copy(inp_hbm.at[idx_v.at[c]], rows_v, s1)
            b = pltpu.async_copy(gtm_hbm.at[idx_v.at[c]], g_v, s2)
            a.wait()
            b.wait()
            pltpu.sync_copy(rows_v, xg_hbm.at[pl.ds(base, CHUNK)])
            pltpu.sync_copy(g_v, gg_hbm.at[pl.ds(base, CHUNK)])

    return k(inp_tm, gates_tm, idx3)


def _expert_body(xg_ref, gg_ref, w_ref, y_ref):
    q = (lax.broadcasted_iota(jnp.int32, (CAP, 128), 0) // 4
         == lax.broadcasted_iota(jnp.int32, (CAP, 128), 1)).astype(jnp.float32)
    scale = jnp.sum(gg_ref[...] * q, axis=1, keepdims=True)
    y_ref[...] = jnp.dot(xg_ref[...] * scale, w_ref[0],
                         preferred_element_type=jnp.float32)


def _expert_mm(xg, gg, expert_w):
    return pl.pallas_call(
        _expert_body,
        grid=(E,),
        in_specs=[
            pl.BlockSpec((CAP, F), lambda e: (e, 0)),
            pl.BlockSpec((CAP, 128), lambda e: (e, 0)),
            pl.BlockSpec((1, F, F), lambda e: (e, 0, 0)),
        ],
        out_specs=pl.BlockSpec((CAP, F), lambda e: (e, 0)),
        out_shape=jax.ShapeDtypeStruct((T, F), jnp.float32),
    )(xg, gg, expert_w)


def _scatter(y, idx3):
    mesh = plsc.VectorSubcoreMesh(core_axis_name="c", subcore_axis_name="s")

    @functools.partial(
        pl.kernel, mesh=mesh,
        out_type=jax.ShapeDtypeStruct((T, F), jnp.float32),
        scratch_types=[pltpu.VMEM((NCH, CHUNK), jnp.int32),
                       pltpu.VMEM((CHUNK, F), jnp.float32),
                       pltpu.SemaphoreType.DMA],
    )
    def k(y_hbm, idx_hbm, out_hbm, idx_v, rows_v, sem):
        wid = lax.axis_index("s") * 2 + lax.axis_index("c")
        pltpu.sync_copy(idx_hbm.at[wid], idx_v)
        for c in range(NCH):
            base = wid * RPW + c * CHUNK
            pltpu.sync_copy(y_hbm.at[pl.ds(base, CHUNK)], rows_v)
            pltpu.async_copy(rows_v, out_hbm.at[idx_v.at[c]], sem).wait()

    return k(y, idx3)


def _untrans_body(y_ref, out_ref):
    out_ref[0] = _trb(y_ref[0])


def _untranspose(y_tm):
    return pl.pallas_call(
        _untrans_body,
        grid=(BATCH, NBLK),
        in_specs=[pl.BlockSpec((1, SEQ_BLK, F), lambda b, j: (b * NBLK + j, 0, 0))],
        out_specs=pl.BlockSpec((1, F, SEQ_BLK), lambda b, j: (b, 0, j)),
        out_shape=jax.ShapeDtypeStruct((BATCH, F, S), jnp.float32),
    )(y_tm.reshape(BATCH * NBLK, SEQ_BLK, F))


def kernel(x, gate, expert_w):
    inp_tm = jnp.transpose(x.astype(jnp.float32), (0, 2, 1)).reshape(T, F)
    logits = inp_tm @ gate.astype(jnp.float32)
    gates = jax.nn.softmax(logits, axis=1)
    gates_pad = jnp.pad(gates, ((0, 0), (0, 128 - E)))
    idx = _route(jnp.transpose(gates))
    idx3 = idx.reshape(NW, NCH, CHUNK)
    xg, gg = _gather(inp_tm, gates_pad, idx3)
    y = _expert_mm(xg, gg, expert_w.astype(jnp.float32))
    y_tm = _scatter(y, idx3)
    return _untranspose(y_tm)
